# baseline (speedup 1.0000x reference)
import jax
import jax.numpy as jnp
from jax.experimental import pallas as pl
from jax.experimental.pallas import tpu as pltpu


def kernel(pixel_values, row_weight, col_weight):
    if pixel_values.ndim != 4:
        raise ValueError('pixel_values must be a 4D tensor')
    b, h, w, _ = pixel_values.shape
    dr = row_weight.shape[1]
    dc = col_weight.shape[1]
    d = dc + dr

    row_w = row_weight[:h]
    col_w = col_weight[:w]

    chunks = []
    pos, size = 0, 2
    while pos < h:
        size = min(size, h - pos)
        chunks.append((pos, size))
        pos += size
        size *= 2

    def body(col_ref, row_ref, out_hbm, tile, sem):
        cw = col_ref[...]
        copies = []
        for k0, kn in chunks:
            rw = row_ref[pl.ds(k0, kn), :]
            tile[pl.ds(k0, kn), :, :dc] = jnp.broadcast_to(
                cw[None, :, :], (kn, w, dc))
            tile[pl.ds(k0, kn), :, dc:] = jnp.broadcast_to(
                rw[:, None, :], (kn, w, dr))
            for ib in range(b):
                c = pltpu.make_async_copy(
                    tile.at[pl.ds(k0, kn)],
                    out_hbm.at[ib, pl.ds(k0, kn)],
                    sem,
                )
                c.start()
                copies.append(c)
        for c in copies:
            c.wait()

    out = pl.pallas_call(
        body,
        in_specs=[
            pl.BlockSpec(memory_space=pltpu.VMEM),
            pl.BlockSpec(memory_space=pltpu.VMEM),
        ],
        out_specs=pl.BlockSpec(memory_space=pl.ANY),
        out_shape=jax.ShapeDtypeStruct((b, h, w, d), jnp.float32),
        scratch_shapes=[
            pltpu.VMEM((h, w, d), jnp.float32),
            pltpu.SemaphoreType.DMA,
        ],
    )(col_w, row_w)
    return out


# speedup vs baseline: 1.0208x; 1.0208x over previous
import jax
import jax.numpy as jnp
from jax.experimental import pallas as pl
from jax.experimental.pallas import tpu as pltpu


def kernel(pixel_values, row_weight, col_weight):
    if pixel_values.ndim != 4:
        raise ValueError('pixel_values must be a 4D tensor')
    b, h, w, _ = pixel_values.shape
    dr = row_weight.shape[1]
    dc = col_weight.shape[1]
    d = dc + dr

    row_w = row_weight[:h]
    col_w = col_weight[:w]

    nchunks = 4 if h % 4 == 0 else 1
    hc = h // nchunks

    def body(col_ref, row_ref, out_hbm, tile, sem):
        cw = col_ref[...]
        copies = []
        for k in range(nchunks):
            rw = row_ref[pl.ds(k * hc, hc), :]
            tile[pl.ds(k * hc, hc), :, :dc] = jnp.broadcast_to(
                cw[None, :, :], (hc, w, dc))
            tile[pl.ds(k * hc, hc), :, dc:] = jnp.broadcast_to(
                rw[:, None, :], (hc, w, dr))
            for ib in range(b):
                c = pltpu.make_async_copy(
                    tile.at[pl.ds(k * hc, hc)],
                    out_hbm.at[ib, pl.ds(k * hc, hc)],
                    sem,
                )
                c.start()
                copies.append(c)
        for c in copies:
            c.wait()

    out = pl.pallas_call(
        body,
        in_specs=[
            pl.BlockSpec(memory_space=pltpu.VMEM),
            pl.BlockSpec(memory_space=pltpu.VMEM),
        ],
        out_specs=pl.BlockSpec(memory_space=pl.ANY),
        out_shape=jax.ShapeDtypeStruct((b, h, w, d), jnp.float32),
        scratch_shapes=[
            pltpu.VMEM((h, w, d), jnp.float32),
            pltpu.SemaphoreType.DMA,
        ],
    )(col_w, row_w)
    return out
